# baseline (speedup 1.0000x reference)
import functools

import jax
import jax.numpy as jnp
from jax import lax
from jax.experimental import pallas as pl
from jax.experimental.pallas import tpu as pltpu
from jax.experimental.pallas import tpu_sc as plsc

N_CLS = 1000
N_CTX = 16
DIM = 768
SUF = 60
SEQ = 77
LANES = 16
NVEC = DIM // LANES

NW = 32


def _row_copy(dst_ref, dst_row, src_ref, src_row):
    for j in range(NVEC):
        dst_ref[dst_row, pl.ds(j * LANES, LANES)] = (
            src_ref[src_row, pl.ds(j * LANES, LANES)])


def _sc_kernel(ctx, ctxn, pre, pren, suf, sufn, out, outn,
               ctx_v, pre_v, suf_v, blk):
    cid = lax.axis_index("c")
    sid = lax.axis_index("s")
    wid = sid * 2 + cid
    n_own = jnp.where(wid < N_CLS % NW, N_CLS // NW + 1, N_CLS // NW)

    def branch(ctx_h, pre_h, suf_h, out_h):
        pltpu.sync_copy(ctx_h, ctx_v)

        def fill_ctx(r, carry):
            _row_copy(blk, 1 + r, ctx_v, r)
            return carry

        lax.fori_loop(0, N_CTX, fill_ctx, 0)

        def body(i, carry):
            c = wid + i * NW
            pltpu.sync_copy(pre_h.at[c], pre_v)
            pltpu.sync_copy(suf_h.at[c], suf_v)
            _row_copy(blk, 0, pre_v, 0)

            def shift(q, carry2):
                _row_copy(blk, 1 + N_CTX + q, suf_v, q)
                return carry2

            lax.fori_loop(0, SUF, shift, 0)
            pltpu.sync_copy(blk, out_h.at[c])
            return carry

        lax.fori_loop(0, n_own, body, 0)

    branch(ctx, pre, suf, out)
    branch(ctxn, pren, sufn, outn)


def kernel(ctx, ctx_neg, token_prefix, token_prefix_neg, token_suffix,
           token_suffix_neg):
    n_cls = token_prefix.shape[0]
    out_t = jax.ShapeDtypeStruct((n_cls, SEQ, DIM), jnp.float32)
    mesh = plsc.VectorSubcoreMesh(core_axis_name="c", subcore_axis_name="s")
    k = functools.partial(
        pl.kernel,
        mesh=mesh,
        out_type=[out_t, out_t],
        scratch_types=[
            pltpu.VMEM((N_CTX, DIM), jnp.float32),
            pltpu.VMEM((1, DIM), jnp.float32),
            pltpu.VMEM((SUF, DIM), jnp.float32),
            pltpu.VMEM((SEQ, DIM), jnp.float32),
        ],
    )(_sc_kernel)
    prompts, prompts_neg = k(ctx, ctx_neg, token_prefix, token_prefix_neg,
                             token_suffix, token_suffix_neg)
    return (prompts, prompts_neg)


# speedup vs baseline: 1.2518x; 1.2518x over previous
import functools

import jax
import jax.numpy as jnp
from jax import lax
from jax.experimental import pallas as pl
from jax.experimental.pallas import tpu as pltpu
from jax.experimental.pallas import tpu_sc as plsc

N_CLS = 1000
N_CTX = 16
DIM = 768
SUF = 60
SEQ = 77
LANES = 16
NVEC = DIM // LANES

NW = 32
HEAD = 1 + N_CTX + 7
TAIL = SEQ - HEAD


def _row_copy(dst_ref, dst_row, src_ref, src_row):
    for j in range(NVEC):
        dst_ref[dst_row, pl.ds(j * LANES, LANES)] = (
            src_ref[src_row, pl.ds(j * LANES, LANES)])


def _rows_loop(n, dst_ref, dst_base, src_ref, src_base):
    def rows(q, carry):
        _row_copy(dst_ref, dst_base + q, src_ref, src_base + q)
        return carry
    lax.fori_loop(0, n, rows, 0)


def _sc_kernel(ctx, ctxn, pre, pren, suf, sufn, out, outn,
               pre_v, ring0, ring1, rlast, blk_h, blk_t0, blk_t1,
               sem_pre, sem_in0, sem_in1, sem_h, sem_t0, sem_t1):
    cid = lax.axis_index("c")
    sid = lax.axis_index("s")
    wid = sid * 2 + cid
    n_own = jnp.where(wid < N_CLS % NW, N_CLS // NW + 1, N_CLS // NW)

    def chunk(suf_h, c, k, ring, sem):
        return pltpu.make_async_copy(
            suf_h.at[c, pl.ds(8 * k, 8)], ring, sem)

    def last_chunk(suf_h, c):
        return pltpu.make_async_copy(
            suf_h.at[c, pl.ds(56, 4)], rlast, sem_in1)

    def branch(ctx_h, pre_h, suf_h, out_h):
        pltpu.sync_copy(ctx_h, blk_h.at[pl.ds(0, N_CTX)])

        def ctx_shift(r, carry):
            _row_copy(blk_h, N_CTX - r, blk_h, N_CTX - 1 - r)
            return carry
        lax.fori_loop(0, N_CTX, ctx_shift, 0)

        def do_class(c, tref, tsem, wait_head, wait_tail):
            pltpu.make_async_copy(pre_h.at[c], pre_v, sem_pre).start()
            chunk(suf_h, c, 0, ring0, sem_in0).start()
            chunk(suf_h, c, 1, ring1, sem_in1).start()

            @pl.when(wait_head)
            def _():
                pltpu.make_async_copy(
                    blk_h, out_h.at[c, pl.ds(0, HEAD)], sem_h).wait()

            pltpu.make_async_copy(pre_h.at[c], pre_v, sem_pre).wait()
            _row_copy(blk_h, 0, pre_v, 0)

            @pl.when(wait_tail)
            def _():
                pltpu.make_async_copy(
                    tref, out_h.at[c, pl.ds(HEAD, TAIL)], tsem).wait()

            chunk(suf_h, c, 0, ring0, sem_in0).wait()
            _rows_loop(7, blk_h, 1 + N_CTX, ring0, 0)
            _row_copy(tref, 0, ring0, 7)
            pltpu.make_async_copy(
                blk_h, out_h.at[c, pl.ds(0, HEAD)], sem_h).start()
            chunk(suf_h, c, 2, ring0, sem_in0).start()

            def pair(m, carry):
                k1 = 2 * m + 1
                chunk(suf_h, c, k1, ring1, sem_in1).wait()
                _rows_loop(8, tref, 8 * k1 - 7, ring1, 0)

                @pl.when(m == 2)
                def _():
                    last_chunk(suf_h, c).start()

                @pl.when(m < 2)
                def _():
                    chunk(suf_h, c, k1 + 2, ring1, sem_in1).start()

                chunk(suf_h, c, k1 + 1, ring0, sem_in0).wait()
                _rows_loop(8, tref, 8 * k1 + 1, ring0, 0)

                @pl.when(m < 2)
                def _():
                    chunk(suf_h, c, k1 + 3, ring0, sem_in0).start()
                return carry

            lax.fori_loop(0, 3, pair, 0)

            last_chunk(suf_h, c).wait()
            _rows_loop(4, tref, 49, rlast, 0)
            pltpu.make_async_copy(
                tref, out_h.at[c, pl.ds(HEAD, TAIL)], tsem).start()

        def pair_body(j, carry):
            c0 = wid + (2 * j) * NW
            do_class(c0, blk_t0, sem_t0, j >= 1, j >= 1)

            @pl.when(2 * j + 1 < n_own)
            def _():
                do_class(c0 + NW, blk_t1, sem_t1, j >= 0, j >= 1)
            return carry

        lax.fori_loop(0, (n_own + 1) // 2, pair_body, 0)

        pltpu.make_async_copy(
            blk_h, out_h.at[wid, pl.ds(0, HEAD)], sem_h).wait()
        pltpu.make_async_copy(
            blk_t0, out_h.at[wid, pl.ds(HEAD, TAIL)], sem_t0).wait()
        pltpu.make_async_copy(
            blk_t1, out_h.at[wid, pl.ds(HEAD, TAIL)], sem_t1).wait()

    branch(ctx, pre, suf, out)
    branch(ctxn, pren, sufn, outn)


def kernel(ctx, ctx_neg, token_prefix, token_prefix_neg, token_suffix,
           token_suffix_neg):
    n_cls = token_prefix.shape[0]
    out_t = jax.ShapeDtypeStruct((n_cls, SEQ, DIM), jnp.float32)
    mesh = plsc.VectorSubcoreMesh(core_axis_name="c", subcore_axis_name="s")
    k = functools.partial(
        pl.kernel,
        mesh=mesh,
        out_type=[out_t, out_t],
        scratch_types=[
            pltpu.VMEM((1, DIM), jnp.float32),
            pltpu.VMEM((8, DIM), jnp.float32),
            pltpu.VMEM((8, DIM), jnp.float32),
            pltpu.VMEM((4, DIM), jnp.float32),
            pltpu.VMEM((HEAD, DIM), jnp.float32),
            pltpu.VMEM((TAIL, DIM), jnp.float32),
            pltpu.VMEM((TAIL, DIM), jnp.float32),
            pltpu.SemaphoreType.DMA,
            pltpu.SemaphoreType.DMA,
            pltpu.SemaphoreType.DMA,
            pltpu.SemaphoreType.DMA,
            pltpu.SemaphoreType.DMA,
            pltpu.SemaphoreType.DMA,
        ],
    )(_sc_kernel)
    prompts, prompts_neg = k(ctx, ctx_neg, token_prefix, token_prefix_neg,
                             token_suffix, token_suffix_neg)
    return (prompts, prompts_neg)
